# baseline (speedup 1.0000x reference)
import functools

import jax
import jax.numpy as jnp
from jax import lax
from jax.experimental import pallas as pl
from jax.experimental.pallas import tpu as pltpu
from jax.experimental.pallas import tpu_sc as plsc

DROP_PROB = 0.1


@functools.lru_cache(maxsize=None)
def _make_sc_kernel(N, C, B):
    info = plsc.get_sparse_core_info()
    NC, NS, L = info.num_cores, info.num_subcores, info.num_lanes
    NW = NC * NS
    rows_per_w = N // NW
    CHUNK = 1024
    n_chunks = rows_per_w // CHUNK
    assert rows_per_w % CHUNK == 0
    assert C == 2 * L
    keep = 1.0 - DROP_PROB

    mesh = plsc.VectorSubcoreMesh(core_axis_name="c", subcore_axis_name="s")

    @functools.partial(
        pl.kernel,
        mesh=mesh,
        out_type=jax.ShapeDtypeStruct((N * C,), jnp.float32),
        scratch_types=[
            pltpu.VMEM((CHUNK * C,), jnp.float32),
            pltpu.VMEM((CHUNK,), jnp.int32),
            pltpu.VMEM((L,), jnp.float32),
            pltpu.VMEM((L,), jnp.float32),
        ],
    )
    def k(rnd_hbm, bid_hbm, data_hbm, out_hbm, buf, bidv, rndv, tab):
        wid = lax.axis_index("s") * NC + lax.axis_index("c")
        base_row = wid * rows_per_w

        pltpu.sync_copy(rnd_hbm, rndv)
        r = rndv[...]
        y = r + jnp.float32(keep)
        fl = y.astype(jnp.int32).astype(jnp.float32)
        tab[...] = fl / jnp.float32(keep)

        tab_v = tab[...]

        def chunk_body(g, _):
            row0 = base_row + g * CHUNK
            pltpu.sync_copy(data_hbm.at[pl.ds(row0 * C, CHUNK * C)], buf)
            pltpu.sync_copy(bid_hbm.at[pl.ds(row0, CHUNK)], bidv)

            def row_body(j, _):
                off16 = j * L
                bid_vec = bidv[pl.ds(off16, L)]
                s_vec = tab_v.at[bid_vec].get(mode="promise_in_bounds")
                for u in range(L):
                    s = s_vec[u]
                    off = (off16 + u) * C
                    buf[pl.ds(off, L)] = buf[pl.ds(off, L)] * s
                    buf[pl.ds(off + L, L)] = buf[pl.ds(off + L, L)] * s
                return 0

            lax.fori_loop(0, CHUNK // L, row_body, 0)
            pltpu.sync_copy(buf, out_hbm.at[pl.ds(row0 * C, CHUNK * C)])
            return 0

        lax.fori_loop(0, n_chunks, chunk_body, 0)

    return k


def kernel(data, rnd, batch_id, depth, batch_size):
    N, C = data.shape
    B = rnd.shape[0]
    k = _make_sc_kernel(N, C, B)
    out = k(rnd.reshape(B), batch_id, data.reshape(N * C))
    return out.reshape(N, C)


# speedup vs baseline: 1.1975x; 1.1975x over previous
import functools

import jax
import jax.numpy as jnp
from jax import lax
from jax.experimental import pallas as pl
from jax.experimental.pallas import tpu as pltpu
from jax.experimental.pallas import tpu_sc as plsc

DROP_PROB = 0.1


@functools.lru_cache(maxsize=None)
def _make_sc_kernel(N, C, B):
    info = plsc.get_sparse_core_info()
    NC, NS, L = info.num_cores, info.num_subcores, info.num_lanes
    NW = NC * NS
    rows_per_w = N // NW
    CHUNK = 256
    n_chunks = rows_per_w // CHUNK
    assert rows_per_w % CHUNK == 0 and n_chunks % 2 == 0
    assert C == 2 * L
    keep = 1.0 - DROP_PROB

    mesh = plsc.VectorSubcoreMesh(core_axis_name="c", subcore_axis_name="s")

    @functools.partial(
        pl.kernel,
        mesh=mesh,
        out_type=jax.ShapeDtypeStruct((N, C), jnp.float32),
        scratch_types=[
            pltpu.VMEM((CHUNK, C), jnp.float32),
            pltpu.VMEM((CHUNK, C), jnp.float32),
            pltpu.VMEM((CHUNK,), jnp.int32),
            pltpu.VMEM((CHUNK,), jnp.int32),
            pltpu.VMEM((L,), jnp.float32),
            pltpu.SemaphoreType.DMA,
            pltpu.SemaphoreType.DMA,
            pltpu.SemaphoreType.DMA,
            pltpu.SemaphoreType.DMA,
            pltpu.SemaphoreType.DMA,
            pltpu.SemaphoreType.DMA,
        ],
    )
    def k(rnd_hbm, bid_hbm, data_hbm, out_hbm,
          b0, b1, bv0, bv1, rndv,
          sd0, sd1, sb0, sb1, so0, so1):
        wid = lax.axis_index("s") * NC + lax.axis_index("c")
        base = wid * rows_per_w

        bufs, bvs = (b0, b1), (bv0, bv1)
        sds, sbs, sos = (sd0, sd1), (sb0, sb1), (so0, so1)

        pltpu.sync_copy(rnd_hbm, rndv)
        r = rndv[...]
        y = r + jnp.float32(keep)
        fl = y.astype(jnp.int32).astype(jnp.float32)
        tab_v = fl / jnp.float32(keep)

        def start_in(g, slot):
            row0 = base + g * CHUNK
            pltpu.async_copy(data_hbm.at[pl.ds(row0, CHUNK)], bufs[slot],
                             sds[slot])
            pltpu.async_copy(bid_hbm.at[pl.ds(row0, CHUNK)], bvs[slot],
                             sbs[slot])

        def wait_in(slot):
            pltpu.make_async_copy(data_hbm.at[pl.ds(0, CHUNK)], bufs[slot],
                                  sds[slot]).wait()
            pltpu.make_async_copy(bid_hbm.at[pl.ds(0, CHUNK)], bvs[slot],
                                  sbs[slot]).wait()

        def start_out(g, slot):
            row0 = base + g * CHUNK
            pltpu.async_copy(bufs[slot], out_hbm.at[pl.ds(row0, CHUNK)],
                             sos[slot])

        def wait_out(slot):
            pltpu.make_async_copy(bufs[slot], out_hbm.at[pl.ds(0, CHUNK)],
                                  sos[slot]).wait()

        def compute(slot):
            buf, bv = bufs[slot], bvs[slot]
            first = bv[pl.ds(0, L)]
            last = bv[pl.ds(CHUNK - L, L)]
            lo = first[0]
            hi = last[L - 1]

            def uniform(_):
                s_vec = tab_v.at[jnp.full((L,), lo, jnp.int32)].get(
                    mode="promise_in_bounds")

                def body(j, _):
                    for u in range(8):
                        rr = j * 8 + u
                        buf[rr, pl.ds(0, L)] = buf[rr, pl.ds(0, L)] * s_vec
                        buf[rr, pl.ds(L, L)] = buf[rr, pl.ds(L, L)] * s_vec
                    return 0

                lax.fori_loop(0, CHUNK // 8, body, 0)
                return 0

            def mixed(_):
                def body(j, _):
                    off16 = j * L
                    bid_vec = bv[pl.ds(off16, L)]
                    s_vec = tab_v.at[bid_vec].get(mode="promise_in_bounds")
                    for u in range(L):
                        s = s_vec[u]
                        rr = off16 + u
                        buf[rr, pl.ds(0, L)] = buf[rr, pl.ds(0, L)] * s
                        buf[rr, pl.ds(L, L)] = buf[rr, pl.ds(L, L)] * s
                    return 0

                lax.fori_loop(0, CHUNK // L, body, 0)
                return 0

            lax.cond(lo == hi, uniform, mixed, 0)

        start_in(0, 0)

        def pair_body(p, _):
            g0 = p * 2
            for slot in (0, 1):
                g = g0 + slot
                wait_in(slot)

                other = 1 - slot

                @pl.when(g + 1 < n_chunks)
                def _():
                    @pl.when(g >= 1)
                    def _():
                        wait_out(other)

                    start_in(g + 1, other)

                compute(slot)
                start_out(g, slot)

            return 0

        lax.fori_loop(0, n_chunks // 2, pair_body, 0)
        wait_out(0)
        wait_out(1)

    return k


def kernel(data, rnd, batch_id, depth, batch_size):
    N, C = data.shape
    B = rnd.shape[0]
    k = _make_sc_kernel(N, C, B)
    return k(rnd.reshape(B), batch_id, data)
